# Initial kernel scaffold; baseline (speedup 1.0000x reference)
import jax
import jax.numpy as jnp
from jax.experimental import pallas as pl


def kernel(z_prev, z_current, codebook):
    raise NotImplementedError("write your pallas kernel here")


# speedup vs baseline: 2.6572x; 2.6572x over previous
import jax
import jax.numpy as jnp
from jax.experimental import pallas as pl

TAU_CONVERGE = 8.0


def _snap_kernel(zp_ref, zc_ref, cb_ref, out_ref):
    TL = zc_ref.shape[0]
    d = zc_ref.shape[-1]
    M = cb_ref.shape[1]
    zc = zc_ref[:, 0, 0, :]
    zp = zp_ref[:, 0, 0, :]
    cb = cb_ref[0]
    diff = zc - zp
    vel = jnp.sqrt(jnp.sum(diff * diff, axis=-1))
    converged = vel < TAU_CONVERGE
    dots = jax.lax.dot_general(cb, zc, (((1,), (1,)), ((), ())),
                               preferred_element_type=jnp.float32)
    c_sq = jnp.sum(cb * cb, axis=-1, keepdims=True)
    dists = c_sq - 2.0 * dots
    mn = jnp.min(dists, axis=0, keepdims=True)
    row = jax.lax.broadcasted_iota(jnp.int32, dists.shape, 0)
    idxv = jnp.min(jnp.where(dists == mn, row, M), axis=0, keepdims=True)
    onehot = (row == idxv).astype(jnp.float32)
    entries = jax.lax.dot_general(onehot, cb, (((0,), (0,)), ((), ())),
                                  preferred_element_type=jnp.float32)
    out_ref[:, 0, 0, :] = jnp.where(converged[:, None], entries, zc)


@jax.jit
def kernel(z_prev, z_current, codebook):
    B, L, dim = z_current.shape
    H, M, d = codebook.shape
    N = B * L
    TL = 512
    zp = z_prev.reshape(N, H, 1, d)
    zc = z_current.reshape(N, H, 1, d)
    out = pl.pallas_call(
        _snap_kernel,
        grid=(H, N // TL),
        in_specs=[
            pl.BlockSpec((TL, 1, 1, d), lambda h, i: (i, h, 0, 0)),
            pl.BlockSpec((TL, 1, 1, d), lambda h, i: (i, h, 0, 0)),
            pl.BlockSpec((1, M, d), lambda h, i: (h, 0, 0)),
        ],
        out_specs=pl.BlockSpec((TL, 1, 1, d), lambda h, i: (i, h, 0, 0)),
        out_shape=jax.ShapeDtypeStruct((N, H, 1, d), jnp.float32),
    )(zp, zc, codebook)
    return out.reshape(B, L, dim)


# speedup vs baseline: 7.7714x; 2.9246x over previous
import jax
import jax.numpy as jnp
from jax.experimental import pallas as pl

TAU_CONVERGE = 8.0


def _snap_kernel(zp_ref, zc_ref, cb_ref, out_ref):
    zc = zc_ref[0]
    zp = zp_ref[0]
    cb = cb_ref[0]
    M = cb.shape[0]
    diff = zc - zp
    vel = jnp.sqrt(jnp.sum(diff * diff, axis=0, keepdims=True))
    converged = vel < TAU_CONVERGE
    dots = jax.lax.dot_general(cb, zc, (((1,), (0,)), ((), ())),
                               preferred_element_type=jnp.float32)
    c_sq = jnp.sum(cb * cb, axis=-1, keepdims=True)
    dists = c_sq - 2.0 * dots
    mn = jnp.min(dists, axis=0, keepdims=True)
    row = jax.lax.broadcasted_iota(jnp.int32, dists.shape, 0)
    idxv = jnp.min(jnp.where(dists == mn, row, M), axis=0, keepdims=True)
    onehot = (row == idxv).astype(jnp.float32)
    entries = jax.lax.dot_general(cb, onehot, (((0,), (0,)), ((), ())),
                                  preferred_element_type=jnp.float32)
    out_ref[0] = jnp.where(converged, entries, zc)


@jax.jit
def kernel(z_prev, z_current, codebook):
    B, L, dim = z_current.shape
    H, M, d = codebook.shape
    N = B * L
    TL = 512
    zp = z_prev.reshape(N, H, d).transpose(1, 2, 0)
    zc = z_current.reshape(N, H, d).transpose(1, 2, 0)
    out = pl.pallas_call(
        _snap_kernel,
        grid=(H, N // TL),
        in_specs=[
            pl.BlockSpec((1, d, TL), lambda h, i: (h, 0, i)),
            pl.BlockSpec((1, d, TL), lambda h, i: (h, 0, i)),
            pl.BlockSpec((1, M, d), lambda h, i: (h, 0, 0)),
        ],
        out_specs=pl.BlockSpec((1, d, TL), lambda h, i: (h, 0, i)),
        out_shape=jax.ShapeDtypeStruct((H, d, N), jnp.float32),
    )(zp, zc, codebook)
    return out.transpose(2, 0, 1).reshape(B, L, dim)


# speedup vs baseline: 10.8933x; 1.4017x over previous
import jax
import jax.numpy as jnp
from jax.experimental import pallas as pl

TAU_CONVERGE = 8.0


def _snap_kernel(zp_ref, zc_ref, cba_ref, out_ref):
    zc = zc_ref[0]
    zp = zp_ref[0]
    cba = cba_ref[0]
    d, TL = zc.shape
    diff = zc - zp
    vel = jnp.sqrt(jnp.sum(diff * diff, axis=0, keepdims=True))
    converged = vel < TAU_CONVERGE
    zca = jnp.concatenate(
        [-2.0 * zc,
         jnp.ones((1, TL), jnp.float32),
         jnp.zeros((1, TL), jnp.float32)], axis=0)
    dists = jax.lax.dot_general(cba, zca, (((1,), (0,)), ((), ())),
                                preferred_element_type=jnp.float32)
    mn = jnp.min(dists, axis=0, keepdims=True)
    onehot = (dists == mn).astype(jnp.float32)
    ent = jax.lax.dot_general(cba, onehot, (((0,), (0,)), ((), ())),
                              preferred_element_type=jnp.float32)
    entries = ent[:d] / ent[d + 1:d + 2]
    out_ref[0] = jnp.where(converged, entries, zc)


@jax.jit
def kernel(z_prev, z_current, codebook):
    B, L, dim = z_current.shape
    H, M, d = codebook.shape
    N = B * L
    TL = 1024
    zp = z_prev.reshape(N, H, d).transpose(1, 2, 0)
    zc = z_current.reshape(N, H, d).transpose(1, 2, 0)
    c_sq = jnp.sum(codebook * codebook, axis=-1, keepdims=True)
    ones = jnp.ones((H, M, 1), jnp.float32)
    cba = jnp.concatenate([codebook, c_sq, ones], axis=-1)
    out = pl.pallas_call(
        _snap_kernel,
        grid=(H, N // TL),
        in_specs=[
            pl.BlockSpec((1, d, TL), lambda h, i: (h, 0, i)),
            pl.BlockSpec((1, d, TL), lambda h, i: (h, 0, i)),
            pl.BlockSpec((1, M, d + 2), lambda h, i: (h, 0, 0)),
        ],
        out_specs=pl.BlockSpec((1, d, TL), lambda h, i: (h, 0, i)),
        out_shape=jax.ShapeDtypeStruct((H, d, N), jnp.float32),
    )(zp, zc, cba)
    return out.transpose(2, 0, 1).reshape(B, L, dim)
